# baseline (device time: 125010 ns/iter reference)
import jax
import jax.numpy as jnp
from jax import lax
from jax.experimental import pallas as pl
from jax.experimental.pallas import tpu as pltpu


def kernel(x, W, labels):
    T, D = x.shape
    _, Vs = W.shape
    NC = 8
    Vc = Vs // NC

    def body(x_ref, w_ref, lab_ref, out_ref,
             l_buf, s128_ref, ll128_ref, acc_ref, recv_ref,
             send_sem, recv_sem):
        i = pl.program_id(0)
        my_x = lax.axis_index("x")
        my_y = lax.axis_index("y")
        my_z = lax.axis_index("z")

        @pl.when(i == 0)
        def _init():
            s128_ref[...] = jnp.zeros_like(s128_ref)
            ll128_ref[...] = jnp.zeros_like(ll128_ref)

        @pl.when(i < NC)
        def _matmul():
            w_bf = w_ref[...].astype(jnp.bfloat16)
            l_buf[i % 2] = lax.dot_general(
                x_ref[...], w_bf,
                (((1,), (0,)), ((), ())),
                preferred_element_type=jnp.float32,
            )

        @pl.when(i > 0)
        def _softmax_prev():
            ic = i - 1
            base = my_x * Vs + ic * Vc
            iota128 = lax.broadcasted_iota(jnp.int32, (T, 128), 1)
            s128 = jnp.zeros((T, 128), jnp.float32)
            ll128 = jnp.zeros((T, 128), jnp.float32)
            for j in range(Vc // 128):
                lsl = l_buf[(i - 1) % 2, :, j * 128:(j + 1) * 128]
                s128 = s128 + jnp.exp(lsl)
                match = iota128 == (lab_ref[...] - (base + j * 128))
                ll128 = ll128 + jnp.where(match, lsl, 0.0)
            s128_ref[...] += s128
            ll128_ref[...] += ll128

        @pl.when(i == NC)
        def _finish():
            acc_ref[:, 0:1] = jnp.sum(s128_ref[...], axis=1, keepdims=True)
            acc_ref[:, 1:2] = jnp.sum(ll128_ref[...], axis=1, keepdims=True)
            partner = (1 - my_x, my_y, my_z)
            barrier = pltpu.get_barrier_semaphore()
            pl.semaphore_signal(
                barrier, inc=1, device_id=partner,
                device_id_type=pl.DeviceIdType.MESH,
            )
            pl.semaphore_wait(barrier, 1)

            rdma = pltpu.make_async_remote_copy(
                src_ref=acc_ref,
                dst_ref=recv_ref,
                send_sem=send_sem,
                recv_sem=recv_sem,
                device_id=partner,
                device_id_type=pl.DeviceIdType.MESH,
            )
            rdma.start()
            rdma.wait()

            s_tot = acc_ref[:, 0:1] + recv_ref[:, 0:1]
            ll_tot = acc_ref[:, 1:2] + recv_ref[:, 1:2]
            out_ref[...] = jnp.log(s_tot) - ll_tot

    lab2d = labels.reshape(T, 1)
    x_bf = x.astype(jnp.bfloat16)

    nll2d = pl.pallas_call(
        body,
        grid=(NC + 1,),
        out_shape=jax.ShapeDtypeStruct((T, 1), jnp.float32),
        in_specs=[
            pl.BlockSpec((T, D), lambda i: (0, 0)),
            pl.BlockSpec((D, Vc), lambda i: (0, jnp.minimum(i, NC - 1))),
            pl.BlockSpec((T, 1), lambda i: (0, 0)),
        ],
        out_specs=pl.BlockSpec((T, 1), lambda i: (0, 0)),
        scratch_shapes=[
            pltpu.VMEM((2, T, Vc), jnp.float32),
            pltpu.VMEM((T, 128), jnp.float32),
            pltpu.VMEM((T, 128), jnp.float32),
            pltpu.VMEM((T, 2), jnp.float32),
            pltpu.VMEM((T, 2), jnp.float32),
            pltpu.SemaphoreType.DMA,
            pltpu.SemaphoreType.DMA,
        ],
        compiler_params=pltpu.CompilerParams(
            collective_id=0,
            dimension_semantics=("arbitrary",),
            vmem_limit_bytes=100 * 1024 * 1024,
        ),
    )(x_bf, W, lab2d)
    return nll2d.reshape(T)


# device time: 121249 ns/iter; 1.0310x vs baseline; 1.0310x over previous
import jax
import jax.numpy as jnp
from jax import lax
from jax.experimental import pallas as pl
from jax.experimental.pallas import tpu as pltpu


def kernel(x, W, labels):
    T, D = x.shape
    _, Vs = W.shape
    NC = 8
    Vc = Vs // NC

    def body(x_ref, w_ref, lab_ref, out_ref,
             la_ref, lb_ref, s128_ref, ll128_ref, acc_ref, recv_ref,
             send_sem, recv_sem):
        i = pl.program_id(0)
        my_x = lax.axis_index("x")
        my_y = lax.axis_index("y")
        my_z = lax.axis_index("z")

        @pl.when(i == 0)
        def _init():
            s128_ref[...] = jnp.zeros_like(s128_ref)
            ll128_ref[...] = jnp.zeros_like(ll128_ref)

        def mm_into(dst_ref):
            w_bf = w_ref[...].astype(jnp.bfloat16)
            dst_ref[...] = lax.dot_general(
                x_ref[...], w_bf,
                (((1,), (0,)), ((), ())),
                preferred_element_type=jnp.float32,
            )

        def softmax_from(src_ref):
            l = src_ref[...]
            e = jnp.exp(l)
            base = my_x * Vs + (i - 1) * Vc
            cols = lax.broadcasted_iota(jnp.int32, (T, Vc), 1) + base
            masked = jnp.where(cols == lab_ref[...], l, 0.0)
            s128 = e[:, 0:128]
            ll128 = masked[:, 0:128]
            for j in range(1, Vc // 128):
                s128 = s128 + e[:, j * 128:(j + 1) * 128]
                ll128 = ll128 + masked[:, j * 128:(j + 1) * 128]
            valid = i > 0
            s128_ref[...] += jnp.where(valid, s128, 0.0)
            ll128_ref[...] += jnp.where(valid, ll128, 0.0)

        @pl.when(lax.rem(i, 2) == 0)
        def _even():
            mm_into(la_ref)
            softmax_from(lb_ref)

        @pl.when(lax.rem(i, 2) == 1)
        def _odd():
            mm_into(lb_ref)
            softmax_from(la_ref)

        @pl.when(i == NC)
        def _finish():
            acc_ref[:, 0:1] = jnp.sum(s128_ref[...], axis=1, keepdims=True)
            acc_ref[:, 1:2] = jnp.sum(ll128_ref[...], axis=1, keepdims=True)
            partner = (1 - my_x, my_y, my_z)
            barrier = pltpu.get_barrier_semaphore()
            pl.semaphore_signal(
                barrier, inc=1, device_id=partner,
                device_id_type=pl.DeviceIdType.MESH,
            )
            pl.semaphore_wait(barrier, 1)

            rdma = pltpu.make_async_remote_copy(
                src_ref=acc_ref,
                dst_ref=recv_ref,
                send_sem=send_sem,
                recv_sem=recv_sem,
                device_id=partner,
                device_id_type=pl.DeviceIdType.MESH,
            )
            rdma.start()
            rdma.wait()

            s_tot = acc_ref[:, 0:1] + recv_ref[:, 0:1]
            ll_tot = acc_ref[:, 1:2] + recv_ref[:, 1:2]
            out_ref[...] = jnp.log(s_tot) - ll_tot

    lab2d = labels.reshape(T, 1)
    x_bf = x.astype(jnp.bfloat16)

    nll2d = pl.pallas_call(
        body,
        grid=(NC + 1,),
        out_shape=jax.ShapeDtypeStruct((T, 1), jnp.float32),
        in_specs=[
            pl.BlockSpec((T, D), lambda i: (0, 0)),
            pl.BlockSpec((D, Vc), lambda i: (0, jnp.minimum(i, NC - 1))),
            pl.BlockSpec((T, 1), lambda i: (0, 0)),
        ],
        out_specs=pl.BlockSpec((T, 1), lambda i: (0, 0)),
        scratch_shapes=[
            pltpu.VMEM((T, Vc), jnp.float32),
            pltpu.VMEM((T, Vc), jnp.float32),
            pltpu.VMEM((T, 128), jnp.float32),
            pltpu.VMEM((T, 128), jnp.float32),
            pltpu.VMEM((T, 2), jnp.float32),
            pltpu.VMEM((T, 2), jnp.float32),
            pltpu.SemaphoreType.DMA,
            pltpu.SemaphoreType.DMA,
        ],
        compiler_params=pltpu.CompilerParams(
            collective_id=0,
            dimension_semantics=("arbitrary",),
            vmem_limit_bytes=100 * 1024 * 1024,
        ),
    )(x_bf, W, lab2d)
    return nll2d.reshape(T)


# device time: 99482 ns/iter; 1.2566x vs baseline; 1.2188x over previous
import jax
import jax.numpy as jnp
from jax import lax
from jax.experimental import pallas as pl
from jax.experimental.pallas import tpu as pltpu


def kernel(x, W, labels):
    T, D = x.shape
    _, Vs = W.shape
    NC = 8
    Vc = Vs // NC

    def body(x_ref, w_ref, lab_ref, out_ref,
             xbf_ref, s128_ref, ll128_ref, acc_ref, recv_ref,
             send_sem, recv_sem):
        i = pl.program_id(0)
        my_x = lax.axis_index("x")
        my_y = lax.axis_index("y")
        my_z = lax.axis_index("z")

        @pl.when(i == 0)
        def _init():
            xbf_ref[...] = x_ref[...].astype(jnp.bfloat16)
            s128_ref[...] = jnp.zeros_like(s128_ref)
            ll128_ref[...] = jnp.zeros_like(ll128_ref)

        w_bf = w_ref[...].astype(jnp.bfloat16)
        l = lax.dot_general(
            xbf_ref[...], w_bf,
            (((1,), (0,)), ((), ())),
            preferred_element_type=jnp.float32,
        )
        e = jnp.exp(l)
        base = my_x * Vs + i * Vc
        cols = lax.broadcasted_iota(jnp.int32, (T, Vc), 1) + base
        masked = jnp.where(cols == lab_ref[...], l, 0.0)
        s128 = e[:, 0:128]
        ll128 = masked[:, 0:128]
        for j in range(1, Vc // 128):
            s128 = s128 + e[:, j * 128:(j + 1) * 128]
            ll128 = ll128 + masked[:, j * 128:(j + 1) * 128]
        s128_ref[...] += s128
        ll128_ref[...] += ll128

        @pl.when(i == NC - 1)
        def _finish():
            acc_ref[:, 0:1] = jnp.sum(s128_ref[...], axis=1, keepdims=True)
            acc_ref[:, 1:2] = jnp.sum(ll128_ref[...], axis=1, keepdims=True)
            partner = (1 - my_x, my_y, my_z)
            barrier = pltpu.get_barrier_semaphore()
            pl.semaphore_signal(
                barrier, inc=1, device_id=partner,
                device_id_type=pl.DeviceIdType.MESH,
            )
            pl.semaphore_wait(barrier, 1)

            rdma = pltpu.make_async_remote_copy(
                src_ref=acc_ref,
                dst_ref=recv_ref,
                send_sem=send_sem,
                recv_sem=recv_sem,
                device_id=partner,
                device_id_type=pl.DeviceIdType.MESH,
            )
            rdma.start()
            rdma.wait()

            s_tot = acc_ref[:, 0:1] + recv_ref[:, 0:1]
            ll_tot = acc_ref[:, 1:2] + recv_ref[:, 1:2]
            out_ref[...] = jnp.log(s_tot) - ll_tot

    lab2d = labels.reshape(T, 1)

    nll2d = pl.pallas_call(
        body,
        grid=(NC,),
        out_shape=jax.ShapeDtypeStruct((T, 1), jnp.float32),
        in_specs=[
            pl.BlockSpec((T, D), lambda i: (0, 0)),
            pl.BlockSpec((D, Vc), lambda i: (0, i)),
            pl.BlockSpec((T, 1), lambda i: (0, 0)),
        ],
        out_specs=pl.BlockSpec((T, 1), lambda i: (0, 0)),
        scratch_shapes=[
            pltpu.VMEM((T, D), jnp.bfloat16),
            pltpu.VMEM((T, 128), jnp.float32),
            pltpu.VMEM((T, 128), jnp.float32),
            pltpu.VMEM((T, 2), jnp.float32),
            pltpu.VMEM((T, 2), jnp.float32),
            pltpu.SemaphoreType.DMA,
            pltpu.SemaphoreType.DMA,
        ],
        compiler_params=pltpu.CompilerParams(
            collective_id=0,
            dimension_semantics=("arbitrary",),
            vmem_limit_bytes=100 * 1024 * 1024,
        ),
    )(x, W, lab2d)
    return nll2d.reshape(T)
